# baseline (speedup 1.0000x reference)
import functools

import jax
import jax.numpy as jnp
from jax import lax
from jax.experimental import pallas as pl
from jax.experimental.pallas import tpu as pltpu
from jax.experimental.pallas import tpu_sc as plsc

_VOCAB = 100000
_EMBED = 128
_B, _T = 1024, 200
_N_TOK = _B * _T
_G = 80
_CHUNKS = 4
_BB = 32


def _sc_gather(table, idx_flat, n_workers):
    n_tok = idx_flat.shape[0]
    tok_per_w = n_tok // n_workers
    groups_per_w = tok_per_w // _G

    mesh = plsc.VectorSubcoreMesh(core_axis_name="c", subcore_axis_name="s")
    nc = mesh.num_cores

    nbuf = 6
    dist = 4
    steady = ((groups_per_w - dist) // nbuf) * nbuf

    @functools.partial(
        pl.kernel,
        out_type=jax.ShapeDtypeStruct((n_tok, _EMBED), jnp.float32),
        mesh=mesh,
        scratch_types=[
            pltpu.VMEM((tok_per_w,), jnp.int32),
            pltpu.VMEM((nbuf, _G, _EMBED), jnp.float32),
        ] + [pltpu.SemaphoreType.DMA] * (2 * nbuf),
    )
    def gather_kernel(table_hbm, idx_hbm, out_hbm, idx_v, rows_v, *sems):
        gsems, ssems = sems[:nbuf], sems[nbuf:]
        wid = lax.axis_index("s") * nc + lax.axis_index("c")
        tbase = wid * tok_per_w
        pltpu.sync_copy(idx_hbm.at[pl.ds(tbase, tok_per_w)], idx_v)

        def gather_cp(j, b):
            return pltpu.make_async_copy(
                table_hbm.at[idx_v.at[pl.ds(j * _G, _G)]], rows_v.at[b],
                gsems[b])

        def scatter_cp(j, b):
            return pltpu.make_async_copy(
                rows_v.at[b], out_hbm.at[pl.ds(tbase + j * _G, _G)], ssems[b])

        for b in range(dist):
            gather_cp(b, b).start()

        def outer(i, carry):
            jj = i * nbuf
            for db in range(nbuf):
                j = jj + db
                bn = (db + dist) % nbuf

                @pl.when(j >= 2)
                def _():
                    scatter_cp(j - 2, bn).wait()

                gather_cp(j + dist, bn).start()
                gather_cp(j, db).wait()
                scatter_cp(j, db).start()
            return carry

        lax.fori_loop(0, steady // nbuf, outer, 0)

        for j in range(steady, groups_per_w):
            if j >= 2:
                scatter_cp(j - 2, (j - 2) % nbuf).wait()
            if j + dist < groups_per_w:
                gather_cp(j + dist, (j + dist) % nbuf).start()
            gather_cp(j, j % nbuf).wait()
            scatter_cp(j, j % nbuf).start()
        for j in range(max(0, groups_per_w - 2), groups_per_w):
            scatter_cp(j, j % nbuf).wait()

    return gather_kernel(table, idx_flat)


def _tc_epilogue_body(*refs):
    if len(refs) == 8:
        refs = refs[1:]
    g_ref, sid_ref, pos_ref, dseg_ref, gamma_ref, beta_ref, o_ref = refs
    base = g_ref[...] + pos_ref[...][None, :, :]
    sidf = sid_ref[...].astype(jnp.float32)
    sidf3 = lax.broadcast_in_dim(sidf, base.shape, (0, 1))
    emb = sidf3 * dseg_ref[0:1, :][None] + base
    bb, t, e = emb.shape
    ones = jnp.ones((e, 1), jnp.float32)
    dn = (((2,), (0,)), ((), ()))
    s1 = lax.dot_general(emb, ones, dn,
                         precision=lax.Precision.DEFAULT,
                         preferred_element_type=jnp.float32)
    s2 = lax.dot_general(emb * emb, ones, dn,
                         precision=lax.Precision.DEFAULT,
                         preferred_element_type=jnp.float32)
    inv_e = 1.0 / e
    mu3 = s1 * inv_e
    var3 = s2 * inv_e - mu3 * mu3
    a3 = lax.rsqrt(var3 + 1e-5) * gamma_ref[0:1, :][None]
    o_ref[...] = (emb - mu3) * a3 + beta_ref[0:1, :][None]


def _tc_epilogue(prev, g3, sid, pos, segtab, gamma, beta, chunk, bc):
    grid = (bc // _BB,)
    base = chunk * (bc // _BB)
    in_specs = [
        pl.BlockSpec((_BB, _T, _EMBED), lambda i: (i, 0, 0)),
        pl.BlockSpec((_BB, _T), lambda i, base=base: (base + i, 0)),
        pl.BlockSpec((_T, _EMBED), lambda i: (0, 0)),
        pl.BlockSpec((1, _EMBED), lambda i: (0, 0)),
        pl.BlockSpec((1, _EMBED), lambda i: (0, 0)),
        pl.BlockSpec((1, _EMBED), lambda i: (0, 0)),
    ]
    args = (g3, sid, pos, segtab, gamma, beta)
    aliases = {}
    if prev is not None:
        in_specs = [pl.BlockSpec(memory_space=pl.ANY)] + in_specs
        args = (prev,) + args
        aliases = {0: 0}
    return pl.pallas_call(
        _tc_epilogue_body,
        grid=grid,
        in_specs=in_specs,
        out_specs=pl.BlockSpec((_BB, _T, _EMBED),
                               lambda i, base=base: (base + i, 0, 0)),
        out_shape=jax.ShapeDtypeStruct((_B, _T, _EMBED), jnp.float32),
        input_output_aliases=aliases,
    )(*args)


def kernel(input_ids, segment_ids, token_table, pos_table, seg_table,
           ln_gamma, ln_beta):
    idx_flat = input_ids.reshape(_N_TOK)
    pos = pos_table[:_T] + seg_table[0]
    dseg = (seg_table[1] - seg_table[0]).reshape(1, _EMBED)
    gamma = ln_gamma.reshape(1, _EMBED)
    beta = ln_beta.reshape(1, _EMBED)

    bc = _B // _CHUNKS
    tc = bc * _T

    gathered = [
        _sc_gather(token_table, lax.slice(idx_flat, (c * tc,),
                                          ((c + 1) * tc,)), 32)
        for c in range(_CHUNKS)
    ]
    out = None
    for c in range(_CHUNKS):
        g3 = gathered[c].reshape(bc, _T, _EMBED)
        out = _tc_epilogue(out, g3, segment_ids, pos, dseg, gamma,
                           beta, c, bc)
    return out


# speedup vs baseline: 1.0010x; 1.0010x over previous
import functools

import jax
import jax.numpy as jnp
from jax import lax
from jax.experimental import pallas as pl
from jax.experimental.pallas import tpu as pltpu
from jax.experimental.pallas import tpu_sc as plsc

_VOCAB = 100000
_EMBED = 128
_B, _T = 1024, 200
_N_TOK = _B * _T
_G = 128
_CHUNKS = 2
_BB = 32


def _sc_gather(table, idx_flat, n_workers):
    n_tok = idx_flat.shape[0]
    tok_per_w = n_tok // n_workers
    groups_per_w = tok_per_w // _G

    mesh = plsc.VectorSubcoreMesh(core_axis_name="c", subcore_axis_name="s")
    nc = mesh.num_cores

    nbuf = 6
    dist = 4
    steady = ((groups_per_w - dist) // nbuf) * nbuf

    @functools.partial(
        pl.kernel,
        out_type=jax.ShapeDtypeStruct((n_tok, _EMBED), jnp.float32),
        mesh=mesh,
        scratch_types=[
            pltpu.VMEM((tok_per_w,), jnp.int32),
            pltpu.VMEM((nbuf, _G, _EMBED), jnp.float32),
        ] + [pltpu.SemaphoreType.DMA] * (2 * nbuf),
    )
    def gather_kernel(table_hbm, idx_hbm, out_hbm, idx_v, rows_v, *sems):
        gsems, ssems = sems[:nbuf], sems[nbuf:]
        wid = lax.axis_index("s") * nc + lax.axis_index("c")
        tbase = wid * tok_per_w
        pltpu.sync_copy(idx_hbm.at[pl.ds(tbase, tok_per_w)], idx_v)

        def gather_cp(j, b):
            return pltpu.make_async_copy(
                table_hbm.at[idx_v.at[pl.ds(j * _G, _G)]], rows_v.at[b],
                gsems[b])

        def scatter_cp(j, b):
            return pltpu.make_async_copy(
                rows_v.at[b], out_hbm.at[pl.ds(tbase + j * _G, _G)], ssems[b])

        for b in range(dist):
            gather_cp(b, b).start()

        def outer(i, carry):
            jj = i * nbuf
            for db in range(nbuf):
                j = jj + db
                bn = (db + dist) % nbuf

                @pl.when(j >= 2)
                def _():
                    scatter_cp(j - 2, bn).wait()

                gather_cp(j + dist, bn).start()
                gather_cp(j, db).wait()
                scatter_cp(j, db).start()
            return carry

        lax.fori_loop(0, steady // nbuf, outer, 0)

        for j in range(steady, groups_per_w):
            if j >= 2:
                scatter_cp(j - 2, (j - 2) % nbuf).wait()
            if j + dist < groups_per_w:
                gather_cp(j + dist, (j + dist) % nbuf).start()
            gather_cp(j, j % nbuf).wait()
            scatter_cp(j, j % nbuf).start()
        for j in range(max(0, groups_per_w - 2), groups_per_w):
            scatter_cp(j, j % nbuf).wait()

    return gather_kernel(table, idx_flat)


def _tc_epilogue_body(*refs):
    if len(refs) == 8:
        refs = refs[1:]
    g_ref, sid_ref, pos_ref, dseg_ref, gamma_ref, beta_ref, o_ref = refs
    base = g_ref[...] + pos_ref[...][None, :, :]
    sidf = sid_ref[...].astype(jnp.float32)
    sidf3 = lax.broadcast_in_dim(sidf, base.shape, (0, 1))
    emb = sidf3 * dseg_ref[0:1, :][None] + base
    bb, t, e = emb.shape
    ones = jnp.ones((e, 1), jnp.float32)
    dn = (((2,), (0,)), ((), ()))
    s1 = lax.dot_general(emb, ones, dn,
                         precision=lax.Precision.DEFAULT,
                         preferred_element_type=jnp.float32)
    s2 = lax.dot_general(emb * emb, ones, dn,
                         precision=lax.Precision.DEFAULT,
                         preferred_element_type=jnp.float32)
    inv_e = 1.0 / e
    mu3 = s1 * inv_e
    var3 = s2 * inv_e - mu3 * mu3
    a3 = lax.rsqrt(var3 + 1e-5) * gamma_ref[0:1, :][None]
    o_ref[...] = (emb - mu3) * a3 + beta_ref[0:1, :][None]


def _tc_epilogue(prev, g3, sid, pos, segtab, gamma, beta, chunk, bc):
    grid = (bc // _BB,)
    base = chunk * (bc // _BB)
    in_specs = [
        pl.BlockSpec((_BB, _T, _EMBED), lambda i: (i, 0, 0)),
        pl.BlockSpec((_BB, _T), lambda i, base=base: (base + i, 0)),
        pl.BlockSpec((_T, _EMBED), lambda i: (0, 0)),
        pl.BlockSpec((1, _EMBED), lambda i: (0, 0)),
        pl.BlockSpec((1, _EMBED), lambda i: (0, 0)),
        pl.BlockSpec((1, _EMBED), lambda i: (0, 0)),
    ]
    args = (g3, sid, pos, segtab, gamma, beta)
    aliases = {}
    if prev is not None:
        in_specs = [pl.BlockSpec(memory_space=pl.ANY)] + in_specs
        args = (prev,) + args
        aliases = {0: 0}
    return pl.pallas_call(
        _tc_epilogue_body,
        grid=grid,
        in_specs=in_specs,
        out_specs=pl.BlockSpec((_BB, _T, _EMBED),
                               lambda i, base=base: (base + i, 0, 0)),
        out_shape=jax.ShapeDtypeStruct((_B, _T, _EMBED), jnp.float32),
        input_output_aliases=aliases,
    )(*args)


def kernel(input_ids, segment_ids, token_table, pos_table, seg_table,
           ln_gamma, ln_beta):
    idx_flat = input_ids.reshape(_N_TOK)
    pos = pos_table[:_T] + seg_table[0]
    dseg = (seg_table[1] - seg_table[0]).reshape(1, _EMBED)
    gamma = ln_gamma.reshape(1, _EMBED)
    beta = ln_beta.reshape(1, _EMBED)

    bc = _B // _CHUNKS
    tc = bc * _T

    gathered = [
        _sc_gather(token_table, lax.slice(idx_flat, (c * tc,),
                                          ((c + 1) * tc,)), 32)
        for c in range(_CHUNKS)
    ]
    out = None
    for c in range(_CHUNKS):
        g3 = gathered[c].reshape(bc, _T, _EMBED)
        out = _tc_epilogue(out, g3, segment_ids, pos, dseg, gamma,
                           beta, c, bc)
    return out


# speedup vs baseline: 1.0433x; 1.0422x over previous
import functools

import jax
import jax.numpy as jnp
from jax import lax
from jax.experimental import pallas as pl
from jax.experimental.pallas import tpu as pltpu
from jax.experimental.pallas import tpu_sc as plsc

_VOCAB = 100000
_EMBED = 128
_B, _T = 1024, 200
_N_TOK = _B * _T
_G = 128
_CHUNKS = 2
_BB = 64


def _sc_gather(table, idx_flat, n_workers):
    n_tok = idx_flat.shape[0]
    tok_per_w = n_tok // n_workers
    groups_per_w = tok_per_w // _G

    mesh = plsc.VectorSubcoreMesh(core_axis_name="c", subcore_axis_name="s")
    nc = mesh.num_cores

    nbuf = 6
    dist = 4
    steady = ((groups_per_w - dist) // nbuf) * nbuf

    @functools.partial(
        pl.kernel,
        out_type=jax.ShapeDtypeStruct((n_tok, _EMBED), jnp.float32),
        mesh=mesh,
        scratch_types=[
            pltpu.VMEM((tok_per_w,), jnp.int32),
            pltpu.VMEM((nbuf, _G, _EMBED), jnp.float32),
        ] + [pltpu.SemaphoreType.DMA] * (2 * nbuf),
    )
    def gather_kernel(table_hbm, idx_hbm, out_hbm, idx_v, rows_v, *sems):
        gsems, ssems = sems[:nbuf], sems[nbuf:]
        wid = lax.axis_index("s") * nc + lax.axis_index("c")
        tbase = wid * tok_per_w
        pltpu.sync_copy(idx_hbm.at[pl.ds(tbase, tok_per_w)], idx_v)

        def gather_cp(j, b):
            return pltpu.make_async_copy(
                table_hbm.at[idx_v.at[pl.ds(j * _G, _G)]], rows_v.at[b],
                gsems[b])

        def scatter_cp(j, b):
            return pltpu.make_async_copy(
                rows_v.at[b], out_hbm.at[pl.ds(tbase + j * _G, _G)], ssems[b])

        for b in range(dist):
            gather_cp(b, b).start()

        def outer(i, carry):
            jj = i * nbuf
            for db in range(nbuf):
                j = jj + db
                bn = (db + dist) % nbuf

                @pl.when(j >= 2)
                def _():
                    scatter_cp(j - 2, bn).wait()

                gather_cp(j + dist, bn).start()
                gather_cp(j, db).wait()
                scatter_cp(j, db).start()
            return carry

        lax.fori_loop(0, steady // nbuf, outer, 0)

        for j in range(steady, groups_per_w):
            if j >= 2:
                scatter_cp(j - 2, (j - 2) % nbuf).wait()
            if j + dist < groups_per_w:
                gather_cp(j + dist, (j + dist) % nbuf).start()
            gather_cp(j, j % nbuf).wait()
            scatter_cp(j, j % nbuf).start()
        for j in range(max(0, groups_per_w - 2), groups_per_w):
            scatter_cp(j, j % nbuf).wait()

    return gather_kernel(table, idx_flat)


def _tc_epilogue_body(*refs):
    if len(refs) == 8:
        refs = refs[1:]
    g_ref, sid_ref, pos_ref, dseg_ref, gamma_ref, beta_ref, o_ref = refs
    base = g_ref[...] + pos_ref[...][None, :, :]
    sidf = sid_ref[...].astype(jnp.float32)
    sidf3 = lax.broadcast_in_dim(sidf, base.shape, (0, 1))
    emb = sidf3 * dseg_ref[0:1, :][None] + base
    mean = jnp.mean(emb, axis=-1, keepdims=True)
    cent = emb - mean
    var = jnp.mean(cent * cent, axis=-1, keepdims=True)
    o_ref[...] = (cent * lax.rsqrt(var + 1e-5) * gamma_ref[0:1, :][None]
                  + beta_ref[0:1, :][None])


def _tc_epilogue(prev, g3, sid, pos, segtab, gamma, beta, chunk, bc):
    grid = (bc // _BB,)
    base = chunk * (bc // _BB)
    in_specs = [
        pl.BlockSpec((_BB, _T, _EMBED), lambda i: (i, 0, 0)),
        pl.BlockSpec((_BB, _T), lambda i, base=base: (base + i, 0)),
        pl.BlockSpec((_T, _EMBED), lambda i: (0, 0)),
        pl.BlockSpec((1, _EMBED), lambda i: (0, 0)),
        pl.BlockSpec((1, _EMBED), lambda i: (0, 0)),
        pl.BlockSpec((1, _EMBED), lambda i: (0, 0)),
    ]
    args = (g3, sid, pos, segtab, gamma, beta)
    aliases = {}
    if prev is not None:
        in_specs = [pl.BlockSpec(memory_space=pl.ANY)] + in_specs
        args = (prev,) + args
        aliases = {0: 0}
    return pl.pallas_call(
        _tc_epilogue_body,
        grid=grid,
        in_specs=in_specs,
        out_specs=pl.BlockSpec((_BB, _T, _EMBED),
                               lambda i, base=base: (base + i, 0, 0)),
        out_shape=jax.ShapeDtypeStruct((_B, _T, _EMBED), jnp.float32),
        input_output_aliases=aliases,
    )(*args)


def kernel(input_ids, segment_ids, token_table, pos_table, seg_table,
           ln_gamma, ln_beta):
    idx_flat = input_ids.reshape(_N_TOK)
    pos = pos_table[:_T] + seg_table[0]
    dseg = (seg_table[1] - seg_table[0]).reshape(1, _EMBED)
    gamma = ln_gamma.reshape(1, _EMBED)
    beta = ln_beta.reshape(1, _EMBED)

    bc = _B // _CHUNKS
    tc = bc * _T

    gathered = [
        _sc_gather(token_table, lax.slice(idx_flat, (c * tc,),
                                          ((c + 1) * tc,)), 32)
        for c in range(_CHUNKS)
    ]
    out = None
    for c in range(_CHUNKS):
        g3 = gathered[c].reshape(bc, _T, _EMBED)
        out = _tc_epilogue(out, g3, segment_ids, pos, dseg, gamma,
                           beta, c, bc)
    return out
